# baseline (speedup 1.0000x reference)
import functools

import jax
import jax.numpy as jnp
from jax import lax
from jax.experimental import pallas as pl
from jax.experimental.pallas import tpu as pltpu
from jax.experimental.pallas import tpu_sc as plsc

B, S, D = 4, 8192, 1024
NC, NS = 2, 16
NW = NC * NS
SEQ_PER_W = S // NW
C = 32


def _sc_kernel(x_hbm, pos_hbm, out_hbm, pos_v, x_v):
    wid = lax.axis_index("s") * NC + lax.axis_index("c")
    base = wid * SEQ_PER_W

    def chunk_body(ci, carry):
        row0 = base + ci * C
        pltpu.sync_copy(pos_hbm.at[pl.ds(row0, C), :], pos_v)
        for b in range(B):
            pltpu.sync_copy(x_hbm.at[b, pl.ds(row0, C), :], x_v)

            def add_row(r, carry2):
                def add_vec(j, carry3):
                    col = j * 16
                    x_v[r, pl.ds(col, 16)] = (
                        x_v[r, pl.ds(col, 16)] + pos_v[r, pl.ds(col, 16)]
                    )
                    return carry3

                return lax.fori_loop(0, D // 16, add_vec, carry2, unroll=8)

            lax.fori_loop(0, C, add_row, 0)
            pltpu.sync_copy(x_v, out_hbm.at[b, pl.ds(row0, C), :])
        return carry

    lax.fori_loop(0, SEQ_PER_W // C, chunk_body, 0)


def kernel(x, pos_table):
    mesh = plsc.VectorSubcoreMesh(core_axis_name="c", subcore_axis_name="s")
    k = functools.partial(
        pl.kernel,
        mesh=mesh,
        out_type=jax.ShapeDtypeStruct((B, S, D), jnp.float32),
        scratch_types=[
            pltpu.VMEM((C, D), jnp.float32),
            pltpu.VMEM((C, D), jnp.float32),
        ],
    )(_sc_kernel)
    return k(x, pos_table)


# speedup vs baseline: 4.9062x; 4.9062x over previous
import jax
import jax.numpy as jnp
from jax.experimental import pallas as pl
from jax.experimental.pallas import tpu as pltpu


def _add_kernel(x_ref, pos_ref, o_ref):
    o_ref[...] = x_ref[...] + pos_ref[...]


def kernel(x, pos_table):
    B, S, D = x.shape
    BS = 2048
    grid = (S // BS, B)
    return pl.pallas_call(
        _add_kernel,
        grid=grid,
        compiler_params=pltpu.CompilerParams(
            dimension_semantics=("parallel", "arbitrary"),
        ),
        in_specs=[
            pl.BlockSpec((1, BS, D), lambda s, b: (b, s, 0)),
            pl.BlockSpec((BS, D), lambda s, b: (s, 0)),
        ],
        out_specs=pl.BlockSpec((1, BS, D), lambda s, b: (b, s, 0)),
        out_shape=jax.ShapeDtypeStruct(x.shape, x.dtype),
    )(x, pos_table)
